# baseline (speedup 1.0000x reference)
import functools

import jax
import jax.numpy as jnp
from jax import lax
from jax.experimental import pallas as pl
from jax.experimental.pallas import tpu as pltpu
from jax.experimental.pallas import tpu_sc as plsc

_DEPTH = 32
_PAD = 128
_CHUNK = 128
_NCH = 8
_UNROLL = 8


def _compact_chunk(rows_ref, comp_ref):
    def step(i, carry):
        for r in range(_UNROLL):
            row = i * _UNROLL + r
            comp_ref[row, pl.ds(0, 16)] = rows_ref[row, pl.ds(0, 16)]
            comp_ref[row, pl.ds(16, 16)] = rows_ref[row, pl.ds(16, 16)]
        return carry

    lax.fori_loop(0, _CHUNK // _UNROLL, step, 0, unroll=False)


def _gather_body(tablep_hbm, x_hbm, out_hbm, idx_v, rows0, rows1, comp0,
                 comp1, sem_i, sg0, sg1, ss0, ss1, *, spans_per_b):
    rows = (rows0, rows1)
    comp = (comp0, comp1)
    sg = (sg0, sg1)
    ss = (ss0, ss1)
    wid = lax.axis_index("s") * 2 + lax.axis_index("c")
    b = wid // spans_per_b
    t0 = (wid % spans_per_b) * (_NCH * _CHUNK)

    idx_copies = [
        pltpu.async_copy(
            x_hbm.at[b, pl.ds(t0 + j * _CHUNK, _CHUNK)], idx_v.at[j], sem_i)
        for j in range(_NCH)
    ]
    for c in idx_copies:
        c.wait()

    gathers = [None] * _NCH
    stores = [None] * _NCH

    def start_gather(j):
        gathers[j] = pltpu.async_copy(
            tablep_hbm.at[idx_v.at[j]], rows[j % 2], sg[j % 2])

    start_gather(0)
    start_gather(1)
    for j in range(_NCH):
        gathers[j].wait()
        if j >= 2:
            stores[j - 2].wait()
        _compact_chunk(rows[j % 2], comp[j % 2])
        if j + 2 < _NCH:
            start_gather(j + 2)
        stores[j] = pltpu.async_copy(
            comp[j % 2],
            out_hbm.at[b, pl.ds(t0 + j * _CHUNK, _CHUNK), :],
            ss[j % 2],
        )
    stores[_NCH - 2].wait()
    stores[_NCH - 1].wait()


@jax.jit
def kernel(x, byte2dsn):
    b, t = x.shape
    spans_per_b = t // (_NCH * _CHUNK)
    x = x.astype(jnp.int32)
    tablep = jnp.pad(byte2dsn, ((0, 0), (0, _PAD - _DEPTH)))

    mesh = plsc.VectorSubcoreMesh(core_axis_name="c", subcore_axis_name="s")
    gather = pl.kernel(
        functools.partial(_gather_body, spans_per_b=spans_per_b),
        mesh=mesh,
        out_type=jax.ShapeDtypeStruct((b, t, _DEPTH), jnp.float32),
        scratch_types=[
            pltpu.VMEM((_NCH, _CHUNK), jnp.int32),
            pltpu.VMEM((_CHUNK, _PAD), jnp.float32),
            pltpu.VMEM((_CHUNK, _PAD), jnp.float32),
            pltpu.VMEM((_CHUNK, _DEPTH), jnp.float32),
            pltpu.VMEM((_CHUNK, _DEPTH), jnp.float32),
        ] + [pltpu.SemaphoreType.DMA] * 5,
        compiler_params=pltpu.CompilerParams(
            use_tc_tiling_on_sc=True,
            disable_bounds_checks=True,
            disable_semaphore_checks=True,
            skip_device_barrier=True,
        ),
    )
    return gather(tablep, x)


# speedup vs baseline: 1.2683x; 1.2683x over previous
import functools

import jax
import jax.numpy as jnp
from jax import lax
from jax.experimental import pallas as pl
from jax.experimental.pallas import tpu as pltpu
from jax.experimental.pallas import tpu_sc as plsc

_DEPTH = 32
_PAD = 128
_SPAN = 1024
_LANES = 16


def _gather_body(tablep_hbm, x_hbm, out_hbm, idx_v, table_v, vals_v, *,
                 spans_per_b):
    wid = lax.axis_index("s") * 2 + lax.axis_index("c")
    b = wid // spans_per_b
    t0 = (wid % spans_per_b) * _SPAN

    pltpu.sync_copy(tablep_hbm, table_v)
    pltpu.sync_copy(x_hbm.at[b, pl.ds(t0, _SPAN)], idx_v)

    col_ids = [jnp.full((_LANES,), d, jnp.int32) for d in range(_DEPTH)]

    def group(g, carry):
        i0 = g * _LANES
        idx16 = idx_v[pl.ds(i0, _LANES)]
        for d in range(_DEPTH):
            vals_v[d, pl.ds(i0, _LANES)] = plsc.load_gather(
                table_v, [idx16, col_ids[d]])
        return carry

    lax.fori_loop(0, _SPAN // _LANES, group, 0, unroll=False)

    pltpu.sync_copy(vals_v, out_hbm.at[b, :, pl.ds(t0, _SPAN)])


@jax.jit
def kernel(x, byte2dsn):
    b, t = x.shape
    spans_per_b = t // _SPAN
    x = x.astype(jnp.int32)
    tablep = jnp.pad(byte2dsn, ((0, 0), (0, _PAD - _DEPTH)))

    mesh = plsc.VectorSubcoreMesh(core_axis_name="c", subcore_axis_name="s")
    gather = pl.kernel(
        functools.partial(_gather_body, spans_per_b=spans_per_b),
        mesh=mesh,
        out_type=jax.ShapeDtypeStruct((b, _DEPTH, t), jnp.float32),
        scratch_types=[
            pltpu.VMEM((_SPAN,), jnp.int32),
            pltpu.VMEM((256, _PAD), jnp.float32),
            pltpu.VMEM((_DEPTH, _SPAN), jnp.float32),
        ],
        compiler_params=pltpu.CompilerParams(
            use_tc_tiling_on_sc=True,
            needs_layout_passes=False,
            disable_bounds_checks=True,
            disable_semaphore_checks=True,
            skip_device_barrier=True,
        ),
    )
    out_t = gather(tablep, x)
    return jnp.transpose(out_t, (0, 2, 1))


# speedup vs baseline: 1.6378x; 1.2913x over previous
import functools

import jax
import jax.numpy as jnp
from jax import lax
from jax.experimental import pallas as pl
from jax.experimental.pallas import tpu as pltpu
from jax.experimental.pallas import tpu_sc as plsc

_DEPTH = 32
_PAD = 128
_SPAN = 1024
_LANES = 16


def _gather_body(tablep_hbm, x_hbm, out_hbm, idx_v, table_v, vals_v, *,
                 spans_per_b):
    wid = lax.axis_index("s") * 2 + lax.axis_index("c")
    b = wid // spans_per_b
    t0 = (wid % spans_per_b) * _SPAN

    pltpu.sync_copy(tablep_hbm, table_v)
    pltpu.sync_copy(x_hbm.at[b, pl.ds(t0, _SPAN)], idx_v)

    col_ids = [jnp.full((_LANES,), d, jnp.int32) for d in range(_DEPTH)]

    @plsc.parallel_loop(0, _SPAN, step=_LANES)
    def group(i0):
        idx16 = idx_v[pl.ds(i0, _LANES)]
        for d in range(_DEPTH):
            vals_v[d, pl.ds(i0, _LANES)] = plsc.load_gather(
                table_v, [idx16, col_ids[d]])

    pltpu.sync_copy(vals_v, out_hbm.at[b, :, pl.ds(t0, _SPAN)])


@jax.jit
def kernel(x, byte2dsn):
    b, t = x.shape
    spans_per_b = t // _SPAN
    x = x.astype(jnp.int32)
    tablep = jnp.pad(byte2dsn, ((0, 0), (0, _PAD - _DEPTH)))

    mesh = plsc.VectorSubcoreMesh(core_axis_name="c", subcore_axis_name="s")
    gather = pl.kernel(
        functools.partial(_gather_body, spans_per_b=spans_per_b),
        mesh=mesh,
        out_type=jax.ShapeDtypeStruct((b, _DEPTH, t), jnp.float32),
        scratch_types=[
            pltpu.VMEM((_SPAN,), jnp.int32),
            pltpu.VMEM((256, _PAD), jnp.float32),
            pltpu.VMEM((_DEPTH, _SPAN), jnp.float32),
        ],
        compiler_params=pltpu.CompilerParams(
            use_tc_tiling_on_sc=True,
            needs_layout_passes=False,
            disable_bounds_checks=True,
            disable_semaphore_checks=True,
            skip_device_barrier=True,
        ),
    )
    out_t = gather(tablep, x)
    return jnp.transpose(out_t, (0, 2, 1))


# speedup vs baseline: 2.4487x; 1.4951x over previous
import functools

import jax
import jax.numpy as jnp
from jax import lax
from jax.experimental import pallas as pl
from jax.experimental.pallas import tpu as pltpu
from jax.experimental.pallas import tpu_sc as plsc

_DEPTH = 32
_SPAN = 1024
_LANES = 16


def _gather_body(tablet_hbm, x_hbm, out_hbm, idx_v, tablet_v, vals_v, *,
                 spans_per_b):
    wid = lax.axis_index("s") * 2 + lax.axis_index("c")
    b = wid // spans_per_b
    t0 = (wid % spans_per_b) * _SPAN

    pltpu.sync_copy(tablet_hbm, tablet_v)
    pltpu.sync_copy(x_hbm.at[b, pl.ds(t0, _SPAN)], idx_v)

    row_ids = [jnp.full((_LANES,), d, jnp.int32) for d in range(_DEPTH)]

    @plsc.parallel_loop(0, _SPAN, step=_LANES, unroll=4)
    def group(i0):
        idx16 = idx_v[pl.ds(i0, _LANES)]
        for d in range(_DEPTH):
            vals_v[d, pl.ds(i0, _LANES)] = plsc.load_gather(
                tablet_v, [row_ids[d], idx16])

    pltpu.sync_copy(vals_v, out_hbm.at[b, :, pl.ds(t0, _SPAN)])


@jax.jit
def kernel(x, byte2dsn):
    b, t = x.shape
    spans_per_b = t // _SPAN
    x = x.astype(jnp.int32)
    tablet = jnp.transpose(byte2dsn)

    mesh = plsc.VectorSubcoreMesh(core_axis_name="c", subcore_axis_name="s")
    gather = pl.kernel(
        functools.partial(_gather_body, spans_per_b=spans_per_b),
        mesh=mesh,
        out_type=jax.ShapeDtypeStruct((b, _DEPTH, t), jnp.float32),
        scratch_types=[
            pltpu.VMEM((_SPAN,), jnp.int32),
            pltpu.VMEM((_DEPTH, 256), jnp.float32),
            pltpu.VMEM((_DEPTH, _SPAN), jnp.float32),
        ],
        compiler_params=pltpu.CompilerParams(
            use_tc_tiling_on_sc=True,
            needs_layout_passes=False,
            disable_bounds_checks=True,
            disable_semaphore_checks=True,
            skip_device_barrier=True,
        ),
    )
    out_t = gather(tablet, x)
    return jnp.transpose(out_t, (0, 2, 1))
